# Initial kernel scaffold; baseline (speedup 1.0000x reference)
import functools

import jax
import jax.numpy as jnp
from jax import lax
from jax.experimental import pallas as pl
from jax.experimental.pallas import tpu as pltpu
from jax.experimental.pallas import tpu_sc as plsc

_CH = 128


@functools.lru_cache(maxsize=None)
def _make_agg(n, d, n_chunks):
    cpt = n_chunks // 16
    rpt = n // 16
    mesh = plsc.VectorSubcoreMesh(core_axis_name="c", subcore_axis_name="s")

    def body(t_lo, t_hi, src2d, dst2d, out_lo, out_hi, acc, sidx, didx, rows, gsem):
        cid = lax.axis_index("c")
        sid = lax.axis_index("s")

        def run(table, out):
            pltpu.sync_copy(table.at[pl.ds(sid * rpt, rpt)],
                            acc.at[pl.ds(sid * rpt, rpt)])
            plsc.subcore_barrier()
            pltpu.sync_copy(src2d.at[pl.ds(sid * cpt, cpt)], sidx)
            pltpu.sync_copy(dst2d.at[pl.ds(sid * cpt, cpt)], didx)

            def step(j, carry):
                pltpu.async_copy(table.at[sidx.at[j]], rows, gsem).wait()
                pltpu.sync_copy(rows, acc.at[didx.at[j]], add=True)
                return carry

            lax.fori_loop(0, cpt, step, 0)
            plsc.subcore_barrier()
            pltpu.sync_copy(acc.at[pl.ds(sid * rpt, rpt)],
                            out.at[pl.ds(sid * rpt, rpt)])

        pl.when(cid == 0)(lambda: run(t_lo, out_lo))
        pl.when(cid == 1)(lambda: run(t_hi, out_hi))

    return pl.kernel(
        body,
        out_type=(jax.ShapeDtypeStruct((n, d), jnp.float32),
                  jax.ShapeDtypeStruct((n, d), jnp.float32)),
        mesh=mesh,
        scratch_types=[
            pltpu.VMEM_SHARED((n + 8, d), jnp.float32),
            pltpu.VMEM((cpt, _CH), jnp.int32),
            pltpu.VMEM((cpt, _CH), jnp.int32),
            pltpu.VMEM((_CH, d), jnp.float32),
            pltpu.SemaphoreType.DMA,
        ],
    )


def _row_tile(n):
    for r in (512, 400, 256, 200, 128, 80, 40, 16, 8):
        if n % r == 0:
            return r
    return 1


def _mlp(a_lo, a_hi, x_lo, x_hi, w_a, b_a, w_b, b_b, eps, final_relu):
    n, hd = a_lo.shape
    dout = w_b.shape[1]
    r = _row_tile(n)
    grid = (n // r,)

    def body(al, ah, xl, xh, wa, ba, wb, bb, o_lo, o_hi):
        h0 = jnp.concatenate([al[...] + eps * xl[...],
                              ah[...] + eps * xh[...]], axis=1)
        z = jnp.dot(h0, wa[...], preferred_element_type=jnp.float32) + ba[...]
        z = jnp.maximum(z, 0.0)
        z = jnp.dot(z, wb[...], preferred_element_type=jnp.float32) + bb[...]
        if final_relu:
            z = jnp.maximum(z, 0.0)
        o_lo[...] = z[:, : dout // 2]
        o_hi[...] = z[:, dout // 2:]

    row_spec = lambda cols: pl.BlockSpec((r, cols), lambda i: (i, 0))
    full = lambda arr: pl.BlockSpec(arr.shape, lambda i: (0,) * arr.ndim)
    return pl.pallas_call(
        body,
        grid=grid,
        in_specs=[row_spec(hd), row_spec(hd), row_spec(hd), row_spec(hd),
                  full(w_a), full(b_a), full(w_b), full(b_b)],
        out_specs=[row_spec(dout // 2), row_spec(dout // 2)],
        out_shape=(jax.ShapeDtypeStruct((n, dout // 2), jnp.float32),
                   jax.ShapeDtypeStruct((n, dout // 2), jnp.float32)),
    )(a_lo, a_hi, x_lo, x_hi, w_a, b_a, w_b, b_b)


def _mlp_final(a_lo, a_hi, x_lo, x_hi, w_a, b_a, w_b, b_b, eps):
    n, hd = a_lo.shape
    dout = w_b.shape[1]
    r = _row_tile(n)

    def body(al, ah, xl, xh, wa, ba, wb, bb, o):
        h0 = jnp.concatenate([al[...] + eps * xl[...],
                              ah[...] + eps * xh[...]], axis=1)
        z = jnp.dot(h0, wa[...], preferred_element_type=jnp.float32) + ba[...]
        z = jnp.maximum(z, 0.0)
        o[...] = jnp.dot(z, wb[...], preferred_element_type=jnp.float32) + bb[...]

    row_spec = lambda cols: pl.BlockSpec((r, cols), lambda i: (i, 0))
    full = lambda arr: pl.BlockSpec(arr.shape, lambda i: (0,) * arr.ndim)
    return pl.pallas_call(
        body,
        grid=(n // r,),
        in_specs=[row_spec(hd), row_spec(hd), row_spec(hd), row_spec(hd),
                  full(w_a), full(b_a), full(w_b), full(b_b)],
        out_specs=pl.BlockSpec((r, dout), lambda i: (i, 0)),
        out_shape=jax.ShapeDtypeStruct((n, dout), jnp.float32),
    )(a_lo, a_hi, x_lo, x_hi, w_a, b_a, w_b, b_b)


def kernel(x, edge_index, W1, b1, W2, b2, W3, b3, W4, b4):
    n, f_in = x.shape
    e = edge_index.shape[1]
    src = edge_index[0].astype(jnp.int32)
    dst = edge_index[1].astype(jnp.int32)

    e_pad = -(-e // 2048) * 2048
    pad = e_pad - e
    src2d = jnp.concatenate([src, jnp.zeros((pad,), jnp.int32)]).reshape(-1, _CH)
    dst2d = jnp.concatenate([dst, jnp.full((pad,), n, jnp.int32)]).reshape(-1, _CH)
    n_chunks = e_pad // _CH

    def half_pad(arr):
        return jnp.concatenate([arr, jnp.zeros((8, arr.shape[1]), arr.dtype)])

    hf = f_in // 2
    x_lo, x_hi = half_pad(x[:, :hf]), half_pad(x[:, hf:])
    b1r, b2r, b3r, b4r = (b.reshape(1, -1) for b in (b1, b2, b3, b4))

    a_lo, a_hi = _make_agg(n, hf, n_chunks)(x_lo, x_hi, src2d, dst2d)
    h_lo, h_hi = _mlp(a_lo, a_hi, x_lo[:n], x_hi[:n], W1, b1r, W2, b2r,
                      1e-09, True)
    hh = W2.shape[1] // 2
    g_lo, g_hi = _make_agg(n, hh, n_chunks)(half_pad(h_lo), half_pad(h_hi),
                                            src2d, dst2d)
    out = _mlp_final(g_lo, g_hi, h_lo, h_hi, W3, b3r, W4, b4r, 1e-13)
    return out


# speedup vs baseline: 2.8327x; 2.8327x over previous
import functools

import jax
import jax.numpy as jnp
from jax import lax
from jax.experimental import pallas as pl
from jax.experimental.pallas import tpu as pltpu
from jax.experimental.pallas import tpu_sc as plsc

_CH = 128
_BI = 16


@functools.lru_cache(maxsize=None)
def _make_agg(n_pad, d, n_chunks, edge_split):
    cpt = n_chunks // (32 if edge_split else 16)
    rpt = n_pad // 16
    mesh = plsc.VectorSubcoreMesh(core_axis_name="c", subcore_axis_name="s")

    def body(t_lo, t_hi, src2d, dst2d, out_lo, out_hi, acc, sidx, didx, rows, gsem):
        cid = lax.axis_index("c")
        sid = lax.axis_index("s")

        def run(table, out):
            pltpu.sync_copy(table.at[pl.ds(sid * rpt, rpt)],
                            acc.at[pl.ds(sid * rpt, rpt)])
            plsc.subcore_barrier()
            if edge_split:
                cbase = (cid * 16 + sid) * cpt
            else:
                cbase = sid * cpt

            def blk_body(bk, carry):
                pltpu.sync_copy(src2d.at[pl.ds(cbase + bk * _BI, _BI)], sidx)
                pltpu.sync_copy(dst2d.at[pl.ds(cbase + bk * _BI, _BI)], didx)

                def step(j, c2):
                    pltpu.async_copy(table.at[sidx.at[j]], rows, gsem).wait()
                    pltpu.sync_copy(rows, acc.at[didx.at[j]], add=True)
                    return c2

                lax.fori_loop(0, _BI, step, 0)
                return carry

            lax.fori_loop(0, cpt // _BI, blk_body, 0)
            plsc.subcore_barrier()
            pltpu.sync_copy(acc.at[pl.ds(sid * rpt, rpt)],
                            out.at[pl.ds(sid * rpt, rpt)])

        if edge_split:
            pl.when(cid == 0)(lambda: run(t_lo, out_lo))
            pl.when(cid == 1)(lambda: run(t_lo, out_hi))
        else:
            pl.when(cid == 0)(lambda: run(t_lo, out_lo))
            pl.when(cid == 1)(lambda: run(t_hi, out_hi))

    return pl.kernel(
        body,
        out_type=(jax.ShapeDtypeStruct((n_pad, d), jnp.float32),
                  jax.ShapeDtypeStruct((n_pad, d), jnp.float32)),
        mesh=mesh,
        scratch_types=[
            pltpu.VMEM_SHARED((n_pad, d), jnp.float32),
            pltpu.VMEM((_BI, _CH), jnp.int32),
            pltpu.VMEM((_BI, _CH), jnp.int32),
            pltpu.VMEM((_CH, d), jnp.float32),
            pltpu.SemaphoreType.DMA,
        ],
    )


def _row_tile(n):
    for r in (512, 400, 256, 200, 128, 80, 40, 16, 8):
        if n % r == 0:
            return r
    return 1


def _mlp1(n, a0, a1, x, w_a, b_a, w_b, b_b, eps):
    n_pad = a0.shape[0]
    din = w_a.shape[0]
    dout = w_b.shape[1]
    r = _row_tile(n)
    ce = eps - 1.0

    def body(a0r, a1r, xr, wa, ba, wb, bb, o_lo, o_hi):
        h0 = a0r[...] + a1r[...] + ce * xr[...]
        z = jnp.dot(h0, wa[...], preferred_element_type=jnp.float32) + ba[...]
        z = jnp.maximum(z, 0.0)
        z = jnp.dot(z, wb[...], preferred_element_type=jnp.float32) + bb[...]
        z = jnp.maximum(z, 0.0)
        o_lo[...] = z[:, : dout // 2]
        o_hi[...] = z[:, dout // 2:]

    row_spec = lambda cols: pl.BlockSpec((r, cols), lambda i: (i, 0))
    full = lambda arr: pl.BlockSpec(arr.shape, lambda i: (0,) * arr.ndim)
    return pl.pallas_call(
        body,
        grid=(n // r,),
        in_specs=[row_spec(din), row_spec(din), row_spec(din),
                  full(w_a), full(b_a), full(w_b), full(b_b)],
        out_specs=[row_spec(dout // 2), row_spec(dout // 2)],
        out_shape=(jax.ShapeDtypeStruct((n_pad, dout // 2), jnp.float32),
                   jax.ShapeDtypeStruct((n_pad, dout // 2), jnp.float32)),
    )(a0, a1, x, w_a, b_a, w_b, b_b)


def _mlp2(n, g_lo, g_hi, h_lo, h_hi, w_a, b_a, w_b, b_b, eps):
    dout = w_b.shape[1]
    hd = g_lo.shape[1]
    r = _row_tile(n)

    def body(gl, gh, hl, hh, wa, ba, wb, bb, o):
        h0 = jnp.concatenate([gl[...] + eps * hl[...],
                              gh[...] + eps * hh[...]], axis=1)
        z = jnp.dot(h0, wa[...], preferred_element_type=jnp.float32) + ba[...]
        z = jnp.maximum(z, 0.0)
        o[...] = jnp.dot(z, wb[...], preferred_element_type=jnp.float32) + bb[...]

    row_spec = lambda cols: pl.BlockSpec((r, cols), lambda i: (i, 0))
    full = lambda arr: pl.BlockSpec(arr.shape, lambda i: (0,) * arr.ndim)
    return pl.pallas_call(
        body,
        grid=(n // r,),
        in_specs=[row_spec(hd), row_spec(hd), row_spec(hd), row_spec(hd),
                  full(w_a), full(b_a), full(w_b), full(b_b)],
        out_specs=pl.BlockSpec((r, dout), lambda i: (i, 0)),
        out_shape=jax.ShapeDtypeStruct((n, dout), jnp.float32),
    )(g_lo, g_hi, h_lo, h_hi, w_a, b_a, w_b, b_b)


def kernel(x, edge_index, W1, b1, W2, b2, W3, b3, W4, b4):
    n, f_in = x.shape
    e = edge_index.shape[1]
    src = edge_index[0].astype(jnp.int32)
    dst = edge_index[1].astype(jnp.int32)

    e_pad = -(-e // (32 * 8 * _CH)) * (32 * 8 * _CH)
    n_pad = -(-n // 128) * 128
    pad = e_pad - e
    src2d = jnp.concatenate([src, jnp.zeros((pad,), jnp.int32)]).reshape(-1, _CH)
    dst2d = jnp.concatenate([dst, jnp.full((pad,), n, jnp.int32)]).reshape(-1, _CH)
    n_chunks = e_pad // _CH

    def row_pad(arr):
        return jnp.concatenate([arr, jnp.zeros((n_pad - n, arr.shape[1]), arr.dtype)])

    x_p = row_pad(x)
    b1r, b2r, b3r, b4r = (b.reshape(1, -1) for b in (b1, b2, b3, b4))

    a0, a1 = _make_agg(n_pad, f_in, n_chunks, True)(x_p, x_p, src2d, dst2d)
    h_lo, h_hi = _mlp1(n, a0, a1, x_p, W1, b1r, W2, b2r, 1e-09)
    hh = W2.shape[1] // 2
    g_lo, g_hi = _make_agg(n_pad, hh, n_chunks, False)(h_lo, h_hi, src2d, dst2d)
    out = _mlp2(n, g_lo, g_hi, h_lo, h_hi, W3, b3r, W4, b4r, 1e-13)
    return out


# speedup vs baseline: 3.3485x; 1.1821x over previous
import functools

import jax
import jax.numpy as jnp
from jax import lax
from jax.experimental import pallas as pl
from jax.experimental.pallas import tpu as pltpu
from jax.experimental.pallas import tpu_sc as plsc

_CH = 128
_BI = 40


@functools.lru_cache(maxsize=None)
def _make_agg(n_pad, d, n_chunks, edge_split):
    cpt = n_chunks // (32 if edge_split else 16)
    rpt = n_pad // 16
    mesh = plsc.VectorSubcoreMesh(core_axis_name="c", subcore_axis_name="s")

    def body(t_lo, t_hi, src2d, dst2d, out_lo, out_hi,
             acc, sidx, didx, rows_a, rows_b, gs_a, gs_b, ss_a, ss_b):
        cid = lax.axis_index("c")
        sid = lax.axis_index("s")

        def run(table, out):
            pltpu.sync_copy(table.at[pl.ds(sid * rpt, rpt)],
                            acc.at[pl.ds(sid * rpt, rpt)])
            plsc.subcore_barrier()
            if edge_split:
                cbase = (cid * 16 + sid) * cpt
            else:
                cbase = sid * cpt

            def fire_g(j, rows, sem):
                return pltpu.async_copy(table.at[sidx.at[j]], rows, sem)

            def fire_s(j, rows, sem):
                return pltpu.async_copy(rows, acc.at[didx.at[j]], sem, add=True)

            def blk_body(bk, carry):
                pltpu.sync_copy(src2d.at[pl.ds(cbase + bk * _BI, _BI)], sidx)
                pltpu.sync_copy(dst2d.at[pl.ds(cbase + bk * _BI, _BI)], didx)
                fire_g(0, rows_a, gs_a)
                fire_g(1, rows_b, gs_b)

                def pair(k, c2):
                    a = 2 * k
                    pltpu.make_async_copy(table.at[sidx.at[a]], rows_a, gs_a).wait()
                    fire_s(a, rows_a, ss_a)
                    pltpu.make_async_copy(rows_a, acc.at[didx.at[a]], ss_a).wait()
                    fire_g(a + 2, rows_a, gs_a)
                    pltpu.make_async_copy(table.at[sidx.at[a + 1]], rows_b, gs_b).wait()
                    fire_s(a + 1, rows_b, ss_b)
                    pltpu.make_async_copy(rows_b, acc.at[didx.at[a + 1]], ss_b).wait()
                    fire_g(a + 3, rows_b, gs_b)
                    return c2

                lax.fori_loop(0, _BI // 2 - 1, pair, 0)
                a = _BI - 2
                pltpu.make_async_copy(table.at[sidx.at[a]], rows_a, gs_a).wait()
                fire_s(a, rows_a, ss_a)
                pltpu.make_async_copy(table.at[sidx.at[a + 1]], rows_b, gs_b).wait()
                fire_s(a + 1, rows_b, ss_b)
                pltpu.make_async_copy(rows_a, acc.at[didx.at[a]], ss_a).wait()
                pltpu.make_async_copy(rows_b, acc.at[didx.at[a + 1]], ss_b).wait()
                return carry

            lax.fori_loop(0, cpt // _BI, blk_body, 0)
            plsc.subcore_barrier()
            pltpu.sync_copy(acc.at[pl.ds(sid * rpt, rpt)],
                            out.at[pl.ds(sid * rpt, rpt)])

        if edge_split:
            pl.when(cid == 0)(lambda: run(t_lo, out_lo))
            pl.when(cid == 1)(lambda: run(t_lo, out_hi))
        else:
            pl.when(cid == 0)(lambda: run(t_lo, out_lo))
            pl.when(cid == 1)(lambda: run(t_hi, out_hi))

    return pl.kernel(
        body,
        out_type=(jax.ShapeDtypeStruct((n_pad, d), jnp.float32),
                  jax.ShapeDtypeStruct((n_pad, d), jnp.float32)),
        mesh=mesh,
        scratch_types=[
            pltpu.VMEM_SHARED((n_pad, d), jnp.float32),
            pltpu.VMEM((_BI, _CH), jnp.int32),
            pltpu.VMEM((_BI, _CH), jnp.int32),
            pltpu.VMEM((_CH, d), jnp.float32),
            pltpu.VMEM((_CH, d), jnp.float32),
            pltpu.SemaphoreType.DMA,
            pltpu.SemaphoreType.DMA,
            pltpu.SemaphoreType.DMA,
            pltpu.SemaphoreType.DMA,
        ],
    )


def _row_tile(n):
    for r in (512, 400, 256, 200, 128, 80, 40, 16, 8):
        if n % r == 0:
            return r
    return 1


def _mlp1(n, a0, a1, x, w_a, b_a, w_b, b_b, eps):
    n_pad = a0.shape[0]
    din = w_a.shape[0]
    dout = w_b.shape[1]
    r = _row_tile(n)
    ce = eps - 1.0

    def body(a0r, a1r, xr, wa, ba, wb, bb, o_lo, o_hi):
        h0 = a0r[...] + a1r[...] + ce * xr[...]
        z = jnp.dot(h0, wa[...], preferred_element_type=jnp.float32) + ba[...]
        z = jnp.maximum(z, 0.0)
        z = jnp.dot(z, wb[...], preferred_element_type=jnp.float32) + bb[...]
        z = jnp.maximum(z, 0.0)
        o_lo[...] = z[:, : dout // 2]
        o_hi[...] = z[:, dout // 2:]

    row_spec = lambda cols: pl.BlockSpec((r, cols), lambda i: (i, 0))
    full = lambda arr: pl.BlockSpec(arr.shape, lambda i: (0,) * arr.ndim)
    return pl.pallas_call(
        body,
        grid=(n // r,),
        in_specs=[row_spec(din), row_spec(din), row_spec(din),
                  full(w_a), full(b_a), full(w_b), full(b_b)],
        out_specs=[row_spec(dout // 2), row_spec(dout // 2)],
        out_shape=(jax.ShapeDtypeStruct((n_pad, dout // 2), jnp.float32),
                   jax.ShapeDtypeStruct((n_pad, dout // 2), jnp.float32)),
    )(a0, a1, x, w_a, b_a, w_b, b_b)


def _mlp2(n, g_lo, g_hi, h_lo, h_hi, w_a, b_a, w_b, b_b, eps):
    dout = w_b.shape[1]
    hd = g_lo.shape[1]
    r = _row_tile(n)

    def body(gl, gh, hl, hh, wa, ba, wb, bb, o):
        h0 = jnp.concatenate([gl[...] + eps * hl[...],
                              gh[...] + eps * hh[...]], axis=1)
        z = jnp.dot(h0, wa[...], preferred_element_type=jnp.float32) + ba[...]
        z = jnp.maximum(z, 0.0)
        o[...] = jnp.dot(z, wb[...], preferred_element_type=jnp.float32) + bb[...]

    row_spec = lambda cols: pl.BlockSpec((r, cols), lambda i: (i, 0))
    full = lambda arr: pl.BlockSpec(arr.shape, lambda i: (0,) * arr.ndim)
    return pl.pallas_call(
        body,
        grid=(n // r,),
        in_specs=[row_spec(hd), row_spec(hd), row_spec(hd), row_spec(hd),
                  full(w_a), full(b_a), full(w_b), full(b_b)],
        out_specs=pl.BlockSpec((r, dout), lambda i: (i, 0)),
        out_shape=jax.ShapeDtypeStruct((n, dout), jnp.float32),
    )(g_lo, g_hi, h_lo, h_hi, w_a, b_a, w_b, b_b)


def kernel(x, edge_index, W1, b1, W2, b2, W3, b3, W4, b4):
    n, f_in = x.shape
    e = edge_index.shape[1]
    src = edge_index[0].astype(jnp.int32)
    dst = edge_index[1].astype(jnp.int32)

    e_pad = -(-e // (32 * 8 * _CH)) * (32 * 8 * _CH)
    n_pad = -(-n // 128) * 128
    pad = e_pad - e
    src2d = jnp.concatenate([src, jnp.zeros((pad,), jnp.int32)]).reshape(-1, _CH)
    dst2d = jnp.concatenate([dst, jnp.full((pad,), n, jnp.int32)]).reshape(-1, _CH)
    n_chunks = e_pad // _CH

    def row_pad(arr):
        return jnp.concatenate([arr, jnp.zeros((n_pad - n, arr.shape[1]), arr.dtype)])

    x_p = row_pad(x)
    b1r, b2r, b3r, b4r = (b.reshape(1, -1) for b in (b1, b2, b3, b4))

    a0, a1 = _make_agg(n_pad, f_in, n_chunks, True)(x_p, x_p, src2d, dst2d)
    h_lo, h_hi = _mlp1(n, a0, a1, x_p, W1, b1r, W2, b2r, 1e-09)
    hh = W2.shape[1] // 2
    g_lo, g_hi = _make_agg(n_pad, hh, n_chunks, False)(h_lo, h_hi, src2d, dst2d)
    out = _mlp2(n, g_lo, g_hi, h_lo, h_hi, W3, b3r, W4, b4r, 1e-13)
    return out
